# Initial kernel scaffold; baseline (speedup 1.0000x reference)
import jax
import jax.numpy as jnp
from jax.experimental import pallas as pl


def kernel(input_id_sequence):
    raise NotImplementedError("write your pallas kernel here")


# speedup vs baseline: 1.4160x; 1.4160x over previous
import jax
import jax.numpy as jnp
from jax.experimental import pallas as pl

NUM_POSITIONS = 4


def _body(in_ref, pos_ref, beh_ref):
    b, t = pos_ref.shape
    t_idx = jax.lax.broadcasted_iota(jnp.int32, (b, t), 1)
    pos_ref[...] = (t_idx % NUM_POSITIONS) + 1
    col1 = in_ref[:, 1:2]
    beh_ref[...] = jnp.where(t_idx == 0, 0, col1)


def kernel(input_id_sequence):
    batch, seq_len = input_id_sequence.shape
    out_shape = jax.ShapeDtypeStruct((batch, seq_len), jnp.int32)
    pos, beh = pl.pallas_call(
        _body,
        grid=(),
        in_specs=[pl.BlockSpec((batch, seq_len), lambda: (0, 0))],
        out_specs=[
            pl.BlockSpec((batch, seq_len), lambda: (0, 0)),
            pl.BlockSpec((batch, seq_len), lambda: (0, 0)),
        ],
        out_shape=[out_shape, out_shape],
    )(input_id_sequence)
    return (pos, beh)
